# baseline (speedup 1.0000x reference)
import functools
import math

import numpy as np
import jax
import jax.numpy as jnp
from jax import lax
from jax.experimental import pallas as pl
from jax.experimental.pallas import tpu as pltpu
from jax.experimental.pallas import tpu_sc as plsc

_D = 64
_S = 200
_BB = 128
_L = 16
_GSTRIDE = 128
_NB = 4


def _pe_table():
    position = np.arange(0, _S)[:, None].astype(np.float32)
    div_term = np.exp(
        np.arange(0, _D, 2).astype(np.float32) * (-math.log(10000.0) / _D))
    pe = np.zeros((_S, _D), dtype=np.float32)
    pe[:, 0::2] = np.sin(position * div_term)
    pe[:, 1::2] = np.cos(position * div_term)
    return jnp.asarray(np.broadcast_to(pe[:, :, None], (_S, _D, _L)).copy())


def _build(n_tok):
    nunits = _S
    scale = float(math.sqrt(_D))

    mesh = plsc.VectorSubcoreMesh(core_axis_name="c", subcore_axis_name="s")

    @functools.partial(
        pl.kernel,
        mesh=mesh,
        out_type=jax.ShapeDtypeStruct((_S, _D, n_tok), jnp.float32),
        scratch_types=[
            pltpu.VMEM((_NB, _BB), jnp.int32),
            pltpu.VMEM((_NB, _BB), jnp.int32),
            pltpu.VMEM((_NB, _BB), jnp.int32),
            pltpu.VMEM((_NB, _BB, _GSTRIDE), jnp.float32),
            pltpu.VMEM((2, _D, _BB), jnp.float32),
            pltpu.VMEM((_NB, _D, _L), jnp.float32),
        ]
        + [pltpu.SemaphoreType.DMA] * (3 * _NB + 2),
        compiler_params=pltpu.CompilerParams(
            needs_layout_passes=False,
            disable_bounds_checks=True,
        ),
    )
    def run(table_h, xt_h, pe_h, out_h, idx_v, lin_v, par_v, g_v, o_v, pe_v,
            *sems):
        xsem = sems[0:_NB]
        gsem = sems[_NB:2 * _NB]
        psem = sems[2 * _NB:3 * _NB]
        osem = sems[3 * _NB:]
        ncores = plsc.get_sparse_core_info().num_cores
        wid = lax.axis_index("s") * ncores + lax.axis_index("c")
        col0 = wid * _BB

        def p_start(s, b):
            pltpu.async_copy(pe_h.at[s], pe_v.at[b], psem[b])

        def p_wait(s, b):
            pltpu.make_async_copy(pe_h.at[s], pe_v.at[b], psem[b]).wait()

        def x_slice(s):
            return xt_h.at[s, pl.ds(col0, _BB)]

        def x_start(s, b):
            pltpu.async_copy(x_slice(s), idx_v.at[b], xsem[b])

        def x_wait(s, b):
            pltpu.make_async_copy(x_slice(s), idx_v.at[b], xsem[b]).wait()

        def g_dst(b):
            if _GSTRIDE == 128:
                return g_v.at[b]
            return g_v.at[b, :, pl.ds(0, 128)]

        def g_start(s, b):
            pltpu.async_copy(table_h.at[lin_v.at[b]], g_dst(b), gsem[b])

        def g_wait(s, b):
            pltpu.make_async_copy(
                table_h.at[lin_v.at[b]], g_dst(b), gsem[b]).wait()

        def o_slice(s):
            return out_h.at[s, :, pl.ds(col0, _BB)]

        def o_start(s, b):
            pltpu.async_copy(o_v.at[b], o_slice(s), osem[b])

        def o_wait(s, b):
            pltpu.make_async_copy(o_v.at[b], o_slice(s), osem[b]).wait()

        def idxcomp(b):
            for g in range(_BB // _L):
                sl = pl.ds(g * _L, _L)
                v = idx_v[b, sl]
                lin_v[b, sl] = lax.shift_right_logical(v, 1)
                par_v[b, sl] = lax.shift_left(lax.bitwise_and(v, 1), 6)

        rows = [lax.iota(jnp.int32, _L) + g * _L for g in range(_BB // _L)]

        def maincomp(s, b, ob):
            pars = [par_v[b, pl.ds(g * _L, _L)] for g in range(_BB // _L)]

            @plsc.parallel_loop(0, _D, 1, unroll=8)
            def d_body(d):
                pe_s = pe_v[b, d, :]
                for g in range(_BB // _L):
                    val = plsc.load_gather(g_v.at[b], [rows[g], pars[g] + d])
                    o_v[ob, d, pl.ds(g * _L, _L)] = val * scale + pe_s

        def step(s, b, ob, *, pre_g, pre_x, owait):
            g_wait(s, b)
            if pre_g:
                b2 = (b + 2) % _NB
                x_wait(s + 2, b2)
                idxcomp(b2)
                g_start(s + 2, b2)
                p_start(s + 2, b2)
            if pre_x:
                x_start(s + 4, b)
            if owait:
                o_wait(s - 2, ob)
            p_wait(s, b)
            maincomp(s, b, ob)
            o_start(s, ob)

        for s in range(4):
            x_start(s, s)
        for s in range(2):
            x_wait(s, s)
            idxcomp(s)
            g_start(s, s)
            p_start(s, s)

        step(0, 0, 0, pre_g=True, pre_x=True, owait=False)
        step(1, 1, 1, pre_g=True, pre_x=True, owait=False)

        def loop_body(k, carry):
            s0 = 2 + k * 4
            for j in range(4):
                step(s0 + j, (2 + j) % _NB, j % 2,
                     pre_g=True, pre_x=True, owait=True)
            return carry

        lax.fori_loop(0, (nunits - 8) // 4, loop_body, 0)

        for s in range(nunits - 6, nunits):
            step(s, s % _NB, s % 2, pre_g=(s + 2 < nunits),
                 pre_x=(s + 4 < nunits), owait=True)

        o_wait(nunits - 2, 0)
        o_wait(nunits - 1, 1)

    return run


def kernel(x, token_embedding):
    bsz, seq = x.shape
    xt = jnp.transpose(x.astype(jnp.int32))
    table2 = token_embedding.reshape(-1, 128)
    run = _build(bsz)
    q = run(table2, xt, _pe_table())
    return jnp.transpose(q, (2, 0, 1))


# speedup vs baseline: 1.0174x; 1.0174x over previous
import functools
import math

import numpy as np
import jax
import jax.numpy as jnp
from jax import lax
from jax.experimental import pallas as pl
from jax.experimental.pallas import tpu as pltpu
from jax.experimental.pallas import tpu_sc as plsc

_D = 64
_S = 200
_BB = 128
_L = 16
_GSTRIDE = 128
_NB = 4


def _pe_table():
    position = np.arange(0, _S)[:, None].astype(np.float32)
    div_term = np.exp(
        np.arange(0, _D, 2).astype(np.float32) * (-math.log(10000.0) / _D))
    pe = np.zeros((_S, _D), dtype=np.float32)
    pe[:, 0::2] = np.sin(position * div_term)
    pe[:, 1::2] = np.cos(position * div_term)
    return jnp.asarray(np.broadcast_to(pe[:, :, None], (_S, _D, _L)).copy())


def _build(n_tok):
    nunits = _S
    scale = float(math.sqrt(_D))

    mesh = plsc.VectorSubcoreMesh(core_axis_name="c", subcore_axis_name="s")

    @functools.partial(
        pl.kernel,
        mesh=mesh,
        out_type=jax.ShapeDtypeStruct((_S, _D, n_tok), jnp.float32),
        scratch_types=[
            pltpu.VMEM((_NB, _BB), jnp.int32),
            pltpu.VMEM((_NB, _BB), jnp.int32),
            pltpu.VMEM((_NB, _BB), jnp.int32),
            pltpu.VMEM((_NB, _BB, _GSTRIDE), jnp.float32),
            pltpu.VMEM((2, _D, _BB), jnp.float32),
            pltpu.VMEM((_NB, _D, _L), jnp.float32),
        ]
        + [pltpu.SemaphoreType.DMA] * (3 * _NB + 2),
        compiler_params=pltpu.CompilerParams(
            needs_layout_passes=False,
            disable_bounds_checks=True,
        ),
    )
    def run(table_h, xt_h, pe_h, out_h, idx_v, lin_v, par_v, g_v, o_v, pe_v,
            *sems):
        xsem = sems[0:_NB]
        gsem = sems[_NB:2 * _NB]
        psem = sems[2 * _NB:3 * _NB]
        osem = sems[3 * _NB:]
        ncores = plsc.get_sparse_core_info().num_cores
        wid = lax.axis_index("s") * ncores + lax.axis_index("c")
        col0 = wid * _BB

        def p_start(s, b):
            pltpu.async_copy(pe_h.at[s], pe_v.at[b], psem[b])

        def p_wait(s, b):
            pltpu.make_async_copy(pe_h.at[s], pe_v.at[b], psem[b]).wait()

        def x_slice(s):
            return xt_h.at[s, pl.ds(col0, _BB)]

        def x_start(s, b):
            pltpu.async_copy(x_slice(s), idx_v.at[b], xsem[b])

        def x_wait(s, b):
            pltpu.make_async_copy(x_slice(s), idx_v.at[b], xsem[b]).wait()

        def g_dst(b):
            if _GSTRIDE == 128:
                return g_v.at[b]
            return g_v.at[b, :, pl.ds(0, 128)]

        def g_start(s, b):
            pltpu.async_copy(table_h.at[lin_v.at[b]], g_dst(b), gsem[b])

        def g_wait(s, b):
            pltpu.make_async_copy(
                table_h.at[lin_v.at[b]], g_dst(b), gsem[b]).wait()

        def o_slice(s):
            return out_h.at[s, :, pl.ds(col0, _BB)]

        def o_start(s, b):
            pltpu.async_copy(o_v.at[b], o_slice(s), osem[b])

        def o_wait(s, b):
            pltpu.make_async_copy(o_v.at[b], o_slice(s), osem[b]).wait()

        def idxcomp(b):
            for g in range(_BB // _L):
                sl = pl.ds(g * _L, _L)
                v = idx_v[b, sl]
                lin_v[b, sl] = lax.shift_right_logical(v, 1)
                par_v[b, sl] = lax.shift_left(lax.bitwise_and(v, 1), 6)

        rows = [lax.iota(jnp.int32, _L) + g * _L for g in range(_BB // _L)]

        def maincomp(s, b, ob):
            pars = [par_v[b, pl.ds(g * _L, _L)] for g in range(_BB // _L)]

            @plsc.parallel_loop(0, _D, 1, unroll=4)
            def d_body(d):
                pe_s = pe_v[b, d, :]
                for g in range(_BB // _L):
                    val = plsc.load_gather(g_v.at[b], [rows[g], pars[g] + d])
                    o_v[ob, d, pl.ds(g * _L, _L)] = val * scale + pe_s

        def step(s, b, ob, *, pre_g, pre_x, owait):
            g_wait(s, b)
            if pre_g:
                b2 = (b + 2) % _NB
                x_wait(s + 2, b2)
                idxcomp(b2)
                g_start(s + 2, b2)
                p_start(s + 2, b2)
            if pre_x:
                x_start(s + 4, b)
            if owait:
                o_wait(s - 2, ob)
            p_wait(s, b)
            maincomp(s, b, ob)
            o_start(s, ob)

        for s in range(4):
            x_start(s, s)
        for s in range(2):
            x_wait(s, s)
            idxcomp(s)
            g_start(s, s)
            p_start(s, s)

        step(0, 0, 0, pre_g=True, pre_x=True, owait=False)
        step(1, 1, 1, pre_g=True, pre_x=True, owait=False)

        def loop_body(k, carry):
            s0 = 2 + k * 4
            for j in range(4):
                step(s0 + j, (2 + j) % _NB, j % 2,
                     pre_g=True, pre_x=True, owait=True)
            return carry

        lax.fori_loop(0, (nunits - 8) // 4, loop_body, 0)

        for s in range(nunits - 6, nunits):
            step(s, s % _NB, s % 2, pre_g=(s + 2 < nunits),
                 pre_x=(s + 4 < nunits), owait=True)

        o_wait(nunits - 2, 0)
        o_wait(nunits - 1, 1)

    return run


def kernel(x, token_embedding):
    bsz, seq = x.shape
    xt = jnp.transpose(x.astype(jnp.int32))
    table2 = token_embedding.reshape(-1, 128)
    run = _build(bsz)
    q = run(table2, xt, _pe_table())
    return jnp.transpose(q, (2, 0, 1))
